# Initial kernel scaffold; baseline (speedup 1.0000x reference)
import jax
import jax.numpy as jnp
from jax.experimental import pallas as pl


def kernel(x, edge_index, W1, b1, W2, b2, Wp, bp):
    raise NotImplementedError("write your pallas kernel here")


# speedup vs baseline: 26.0617x; 26.0617x over previous
import functools

import jax
import jax.numpy as jnp
from jax import lax
from jax.experimental import pallas as pl
from jax.experimental.pallas import tpu as pltpu
from jax.experimental.pallas import tpu_sc as plsc

N = 10000
NZ = 10240
CP = 48
F_IN = 128
HID = 256
C = 40
E = 320000
K_HOPS = 10

NW = 32
NB = 80
BB = 128
EPW = NB * BB
E_PAD = NW * EPW
RPT = NZ // 16

_mesh = plsc.VectorSubcoreMesh(core_axis_name="c", subcore_axis_name="s")


@functools.partial(
    pl.kernel,
    out_type=jax.ShapeDtypeStruct((2, NZ), jnp.float32),
    mesh=_mesh,
    compiler_params=pltpu.CompilerParams(use_tc_tiling_on_sc=False),
    scratch_types=[
        pltpu.VMEM((NB, BB), jnp.int32),
        pltpu.VMEM((BB,), jnp.float32),
        pltpu.VMEM_SHARED((NZ,), jnp.float32),
    ],
)
def _deg_kernel(cs_hbm, zeros_hbm, ones_hbm, out_hbm, cidx, ones_v, deg_sp):
    c = lax.axis_index("c")
    s = lax.axis_index("s")
    wid = s * 2 + c
    pltpu.sync_copy(zeros_hbm.at[pl.ds(s * RPT, RPT)], deg_sp.at[pl.ds(s * RPT, RPT)])
    pltpu.sync_copy(cs_hbm.at[wid], cidx)
    pltpu.sync_copy(ones_hbm, ones_v)
    plsc.subcore_barrier()

    def body(j, carry):
        pltpu.sync_copy(ones_v, deg_sp.at[cidx.at[j]], add=True)
        return carry

    lax.fori_loop(0, NB, body, 0)
    plsc.subcore_barrier()
    pltpu.sync_copy(deg_sp.at[pl.ds(s * RPT, RPT)], out_hbm.at[c, pl.ds(s * RPT, RPT)])


@functools.partial(
    pl.kernel,
    out_type=jax.ShapeDtypeStruct((2, NZ, CP), jnp.float32),
    mesh=_mesh,
    compiler_params=pltpu.CompilerParams(use_tc_tiling_on_sc=False),
    scratch_types=[
        pltpu.VMEM((NB, BB), jnp.int32),
        pltpu.VMEM((NB, BB), jnp.int32),
        pltpu.VMEM((BB, CP), jnp.float32),
        pltpu.VMEM((BB, CP), jnp.float32),
        pltpu.VMEM_SHARED((NZ, CP), jnp.float32),
        pltpu.SemaphoreType.DMA,
        pltpu.SemaphoreType.DMA,
    ],
)
def _hop_kernel(y_hbm, rs_hbm, cs_hbm, zeros_hbm, out_hbm,
                ridx, cidx, gbuf0, gbuf1, z_sp, sem0, sem1):
    c = lax.axis_index("c")
    s = lax.axis_index("s")
    wid = s * 2 + c
    pltpu.sync_copy(zeros_hbm.at[pl.ds(s * RPT, RPT)], z_sp.at[pl.ds(s * RPT, RPT)])
    pltpu.sync_copy(rs_hbm.at[wid], ridx)
    pltpu.sync_copy(cs_hbm.at[wid], cidx)
    plsc.subcore_barrier()

    pltpu.async_copy(y_hbm.at[ridx.at[0]], gbuf0, sem0)

    def body(i, carry):
        j0 = 2 * i
        pltpu.async_copy(y_hbm.at[ridx.at[j0 + 1]], gbuf1, sem1)
        pltpu.make_async_copy(y_hbm.at[ridx.at[j0]], gbuf0, sem0).wait()
        pltpu.sync_copy(gbuf0, z_sp.at[cidx.at[j0]], add=True)

        @pl.when(j0 + 2 < NB)
        def _():
            pltpu.async_copy(y_hbm.at[ridx.at[j0 + 2]], gbuf0, sem0)

        pltpu.make_async_copy(y_hbm.at[ridx.at[j0 + 1]], gbuf1, sem1).wait()
        pltpu.sync_copy(gbuf1, z_sp.at[cidx.at[j0 + 1]], add=True)
        return carry

    lax.fori_loop(0, NB // 2, body, 0)
    plsc.subcore_barrier()
    pltpu.sync_copy(z_sp.at[pl.ds(s * RPT, RPT)], out_hbm.at[c, pl.ds(s * RPT, RPT)])


def _dense_body(x_ref, w1_ref, b1_ref, w2_ref, b2_ref, o_ref):
    h = jnp.dot(x_ref[...], w1_ref[...], preferred_element_type=jnp.float32)
    h = jnp.maximum(h + b1_ref[...], 0.0)
    o_ref[...] = jnp.dot(h, w2_ref[...], preferred_element_type=jnp.float32) + b2_ref[...]


def _final_body(pps_ref, wp_ref, bp_ref, o_ref):
    pps = pps_ref[...]
    wp = wp_ref[...]
    bp = bp_ref[0, 0]
    score = jnp.sum(pps * wp[0][None, None, :], axis=2) + bp
    retain = jax.nn.sigmoid(score)
    out = jnp.sum(retain[:, :, None] * pps, axis=0)
    colp = lax.broadcasted_iota(jnp.int32, out.shape, 1)
    logits = jnp.where(colp < C, out, -jnp.inf)
    m = jnp.max(logits, axis=1, keepdims=True)
    lse = jnp.log(jnp.sum(jnp.exp(logits - m), axis=1, keepdims=True))
    o_ref[...] = logits - m - lse


def kernel(x, edge_index, W1, b1, W2, b2, Wp, bp):
    f32 = jnp.float32
    x_pad = jnp.zeros((NZ, F_IN), f32).at[:N].set(x)
    W1t = W1.T
    W2t = jnp.zeros((HID, CP), f32).at[:, :C].set(W2.T)
    b1r = b1.reshape(1, HID)
    b2r = jnp.zeros((1, CP), f32).at[0, :C].set(b2)
    Wpr = jnp.zeros((1, CP), f32).at[0, :C].set(Wp[0])
    bpr = bp.reshape(1, 1)

    rs0 = edge_index[0]
    cs0 = edge_index[1]
    ar = jnp.arange(E, dtype=jnp.int32)
    trash = N + (ar % (NZ - N))
    cs1 = jnp.where(rs0 == cs0, trash, cs0)
    npad = E_PAD - E
    arp = jnp.arange(npad, dtype=jnp.int32)
    rs3 = jnp.concatenate([rs0, arp % N]).reshape(NW, NB, BB)
    cs3 = jnp.concatenate([cs1, N + (arp % (NZ - N))]).reshape(NW, NB, BB)

    zeros2 = jnp.zeros((NZ, CP), f32)
    zeros1 = jnp.zeros((NZ,), f32)
    ones1 = jnp.ones((BB,), f32)

    BD = 512
    h = pl.pallas_call(
        _dense_body,
        grid=(NZ // BD,),
        in_specs=[
            pl.BlockSpec((BD, F_IN), lambda i: (i, 0)),
            pl.BlockSpec((F_IN, HID), lambda i: (0, 0)),
            pl.BlockSpec((1, HID), lambda i: (0, 0)),
            pl.BlockSpec((HID, CP), lambda i: (0, 0)),
            pl.BlockSpec((1, CP), lambda i: (0, 0)),
        ],
        out_specs=pl.BlockSpec((BD, CP), lambda i: (i, 0)),
        out_shape=jax.ShapeDtypeStruct((NZ, CP), f32),
    )(x_pad, W1t, b1r, W2t, b2r)

    degp = _deg_kernel(cs3, zeros1, ones1)
    deg = degp[0] + degp[1] + 1.0
    disc = lax.rsqrt(deg)[:, None]

    preds = [h]
    y = h * disc
    for k in range(K_HOPS):
        zp = _hop_kernel(y, rs3, cs3, zeros2)
        cur = disc * (zp[0] + zp[1] + y)
        preds.append(cur)
        if k + 1 < K_HOPS:
            y = disc * cur

    pps = jnp.stack(preds)
    out = pl.pallas_call(
        _final_body,
        grid=(NZ // BD,),
        in_specs=[
            pl.BlockSpec((K_HOPS + 1, BD, CP), lambda i: (0, i, 0)),
            pl.BlockSpec((1, CP), lambda i: (0, 0)),
            pl.BlockSpec((1, 1), lambda i: (0, 0)),
        ],
        out_specs=pl.BlockSpec((BD, CP), lambda i: (i, 0)),
        out_shape=jax.ShapeDtypeStruct((NZ, CP), f32),
    )(pps, Wpr, bpr)
    return out[:N, :C]


# speedup vs baseline: 27.9932x; 1.0741x over previous
import functools

import jax
import jax.numpy as jnp
from jax import lax
from jax.experimental import pallas as pl
from jax.experimental.pallas import tpu as pltpu
from jax.experimental.pallas import tpu_sc as plsc

N = 10000
NZ = 10240
CP = 48
F_IN = 128
HID = 256
C = 40
E = 320000
K_HOPS = 10

NW = 32
NB = 80
BB = 128
EPW = NB * BB
E_PAD = NW * EPW
RPT = NZ // 16

_mesh = plsc.VectorSubcoreMesh(core_axis_name="c", subcore_axis_name="s")


@functools.partial(
    pl.kernel,
    out_type=jax.ShapeDtypeStruct((2, NZ), jnp.float32),
    mesh=_mesh,
    compiler_params=pltpu.CompilerParams(use_tc_tiling_on_sc=False),
    scratch_types=[
        pltpu.VMEM((NB, BB), jnp.int32),
        pltpu.VMEM((BB,), jnp.float32),
        pltpu.VMEM_SHARED((NZ,), jnp.float32),
    ],
)
def _deg_kernel(cs_hbm, zeros_hbm, ones_hbm, out_hbm, cidx, ones_v, deg_sp):
    c = lax.axis_index("c")
    s = lax.axis_index("s")
    wid = s * 2 + c
    pltpu.sync_copy(zeros_hbm.at[pl.ds(s * RPT, RPT)], deg_sp.at[pl.ds(s * RPT, RPT)])
    pltpu.sync_copy(cs_hbm.at[wid], cidx)
    pltpu.sync_copy(ones_hbm, ones_v)
    plsc.subcore_barrier()

    def body(j, carry):
        pltpu.sync_copy(ones_v, deg_sp.at[cidx.at[j]], add=True)
        return carry

    lax.fori_loop(0, NB, body, 0)
    plsc.subcore_barrier()
    pltpu.sync_copy(deg_sp.at[pl.ds(s * RPT, RPT)], out_hbm.at[c, pl.ds(s * RPT, RPT)])


@functools.partial(
    pl.kernel,
    out_type=jax.ShapeDtypeStruct((2, NZ, CP), jnp.float32),
    mesh=_mesh,
    compiler_params=pltpu.CompilerParams(use_tc_tiling_on_sc=False),
    scratch_types=[
        pltpu.VMEM((NB, BB), jnp.int32),
        pltpu.VMEM((NB, BB), jnp.int32),
        pltpu.VMEM((BB, CP), jnp.float32),
        pltpu.VMEM((BB, CP), jnp.float32),
        pltpu.VMEM((BB, CP), jnp.float32),
        pltpu.VMEM((BB, CP), jnp.float32),
        pltpu.VMEM_SHARED((NZ, CP), jnp.float32),
        pltpu.SemaphoreType.DMA,
        pltpu.SemaphoreType.DMA,
        pltpu.SemaphoreType.DMA,
        pltpu.SemaphoreType.DMA,
        pltpu.SemaphoreType.DMA,
        pltpu.SemaphoreType.DMA,
        pltpu.SemaphoreType.DMA,
        pltpu.SemaphoreType.DMA,
    ],
)
def _hop_kernel(y_hbm, rs_hbm, cs_hbm, zeros_hbm, out_hbm,
                ridx, cidx, gb0, gb1, gb2, gb3, z_sp,
                gs0, gs1, gs2, gs3, ss0, ss1, ss2, ss3):
    c = lax.axis_index("c")
    s = lax.axis_index("s")
    wid = s * 2 + c
    gbufs = (gb0, gb1, gb2, gb3)
    gsems = (gs0, gs1, gs2, gs3)
    ssems = (ss0, ss1, ss2, ss3)
    pltpu.sync_copy(zeros_hbm.at[pl.ds(s * RPT, RPT)], z_sp.at[pl.ds(s * RPT, RPT)])
    pltpu.sync_copy(rs_hbm.at[wid], ridx)
    pltpu.sync_copy(cs_hbm.at[wid], cidx)
    plsc.subcore_barrier()

    pltpu.async_copy(y_hbm.at[ridx.at[0]], gb0, gs0)
    pltpu.async_copy(y_hbm.at[ridx.at[1]], gb1, gs1)

    def group(i, carry):
        j0 = 4 * i
        for b in range(4):
            j = j0 + b

            @pl.when(j >= 2)
            def _():
                pltpu.make_async_copy(gbufs[(b + 2) % 4],
                                      z_sp.at[cidx.at[j - 2]],
                                      ssems[(b + 2) % 4]).wait()

            pltpu.make_async_copy(y_hbm.at[ridx.at[j]], gbufs[b], gsems[b]).wait()
            pltpu.async_copy(gbufs[b], z_sp.at[cidx.at[j]], ssems[b], add=True)

            @pl.when(j + 2 < NB)
            def _():
                pltpu.async_copy(y_hbm.at[ridx.at[j + 2]],
                                 gbufs[(b + 2) % 4], gsems[(b + 2) % 4])
        return carry

    lax.fori_loop(0, NB // 4, group, 0)
    pltpu.make_async_copy(gb2, z_sp.at[cidx.at[NB - 2]], ss2).wait()
    pltpu.make_async_copy(gb3, z_sp.at[cidx.at[NB - 1]], ss3).wait()
    plsc.subcore_barrier()
    pltpu.sync_copy(z_sp.at[pl.ds(s * RPT, RPT)], out_hbm.at[c, pl.ds(s * RPT, RPT)])


def _dense_body(x_ref, w1_ref, b1_ref, w2_ref, b2_ref, o_ref):
    h = jnp.dot(x_ref[...], w1_ref[...], preferred_element_type=jnp.float32)
    h = jnp.maximum(h + b1_ref[...], 0.0)
    o_ref[...] = jnp.dot(h, w2_ref[...], preferred_element_type=jnp.float32) + b2_ref[...]


def _final_body(pps_ref, wp_ref, bp_ref, o_ref):
    pps = pps_ref[...]
    wp = wp_ref[...]
    bp = bp_ref[0, 0]
    score = jnp.sum(pps * wp[0][None, None, :], axis=2) + bp
    retain = jax.nn.sigmoid(score)
    out = jnp.sum(retain[:, :, None] * pps, axis=0)
    colp = lax.broadcasted_iota(jnp.int32, out.shape, 1)
    logits = jnp.where(colp < C, out, -jnp.inf)
    m = jnp.max(logits, axis=1, keepdims=True)
    lse = jnp.log(jnp.sum(jnp.exp(logits - m), axis=1, keepdims=True))
    o_ref[...] = logits - m - lse


def kernel(x, edge_index, W1, b1, W2, b2, Wp, bp):
    f32 = jnp.float32
    x_pad = jnp.zeros((NZ, F_IN), f32).at[:N].set(x)
    W1t = W1.T
    W2t = jnp.zeros((HID, CP), f32).at[:, :C].set(W2.T)
    b1r = b1.reshape(1, HID)
    b2r = jnp.zeros((1, CP), f32).at[0, :C].set(b2)
    Wpr = jnp.zeros((1, CP), f32).at[0, :C].set(Wp[0])
    bpr = bp.reshape(1, 1)

    rs0 = edge_index[0]
    cs0 = edge_index[1]
    ar = jnp.arange(E, dtype=jnp.int32)
    trash = N + (ar % (NZ - N))
    cs1 = jnp.where(rs0 == cs0, trash, cs0)
    npad = E_PAD - E
    arp = jnp.arange(npad, dtype=jnp.int32)
    rs3 = jnp.concatenate([rs0, arp % N]).reshape(NW, NB, BB)
    cs3 = jnp.concatenate([cs1, N + (arp % (NZ - N))]).reshape(NW, NB, BB)

    zeros2 = jnp.zeros((NZ, CP), f32)
    zeros1 = jnp.zeros((NZ,), f32)
    ones1 = jnp.ones((BB,), f32)

    BD = 512
    h = pl.pallas_call(
        _dense_body,
        grid=(NZ // BD,),
        in_specs=[
            pl.BlockSpec((BD, F_IN), lambda i: (i, 0)),
            pl.BlockSpec((F_IN, HID), lambda i: (0, 0)),
            pl.BlockSpec((1, HID), lambda i: (0, 0)),
            pl.BlockSpec((HID, CP), lambda i: (0, 0)),
            pl.BlockSpec((1, CP), lambda i: (0, 0)),
        ],
        out_specs=pl.BlockSpec((BD, CP), lambda i: (i, 0)),
        out_shape=jax.ShapeDtypeStruct((NZ, CP), f32),
    )(x_pad, W1t, b1r, W2t, b2r)

    degp = _deg_kernel(cs3, zeros1, ones1)
    deg = degp[0] + degp[1] + 1.0
    disc = lax.rsqrt(deg)[:, None]

    preds = [h]
    y = h * disc
    for k in range(K_HOPS):
        zp = _hop_kernel(y, rs3, cs3, zeros2)
        cur = disc * (zp[0] + zp[1] + y)
        preds.append(cur)
        if k + 1 < K_HOPS:
            y = disc * cur

    pps = jnp.stack(preds)
    out = pl.pallas_call(
        _final_body,
        grid=(NZ // BD,),
        in_specs=[
            pl.BlockSpec((K_HOPS + 1, BD, CP), lambda i: (0, i, 0)),
            pl.BlockSpec((1, CP), lambda i: (0, 0)),
            pl.BlockSpec((1, 1), lambda i: (0, 0)),
        ],
        out_specs=pl.BlockSpec((BD, CP), lambda i: (i, 0)),
        out_shape=jax.ShapeDtypeStruct((NZ, CP), f32),
    )(pps, Wpr, bpr)
    return out[:N, :C]


# speedup vs baseline: 29.7610x; 1.0632x over previous
import functools

import jax
import jax.numpy as jnp
from jax import lax
from jax.experimental import pallas as pl
from jax.experimental.pallas import tpu as pltpu
from jax.experimental.pallas import tpu_sc as plsc

N = 10000
NZ = 10240
CP = 48
F_IN = 128
HID = 256
C = 40
E = 320000
K_HOPS = 10

NW = 32
NB = 80
BB = 128
EPW = NB * BB
E_PAD = NW * EPW
RPT = NZ // 16

_mesh = plsc.VectorSubcoreMesh(core_axis_name="c", subcore_axis_name="s")


@functools.partial(
    pl.kernel,
    out_type=jax.ShapeDtypeStruct((2, NZ), jnp.float32),
    mesh=_mesh,
    compiler_params=pltpu.CompilerParams(use_tc_tiling_on_sc=False),
    scratch_types=[
        pltpu.VMEM((NB, BB), jnp.int32),
        pltpu.VMEM((BB,), jnp.float32),
        pltpu.VMEM_SHARED((NZ,), jnp.float32),
    ],
)
def _deg_kernel(cs_hbm, zeros_hbm, ones_hbm, out_hbm, cidx, ones_v, deg_sp):
    c = lax.axis_index("c")
    s = lax.axis_index("s")
    wid = s * 2 + c
    pltpu.sync_copy(zeros_hbm.at[pl.ds(s * RPT, RPT)], deg_sp.at[pl.ds(s * RPT, RPT)])
    pltpu.sync_copy(cs_hbm.at[wid], cidx)
    pltpu.sync_copy(ones_hbm, ones_v)
    plsc.subcore_barrier()

    def body(j, carry):
        pltpu.sync_copy(ones_v, deg_sp.at[cidx.at[j]], add=True)
        return carry

    lax.fori_loop(0, NB, body, 0)
    plsc.subcore_barrier()
    pltpu.sync_copy(deg_sp.at[pl.ds(s * RPT, RPT)], out_hbm.at[c, pl.ds(s * RPT, RPT)])


@functools.partial(
    pl.kernel,
    out_type=jax.ShapeDtypeStruct((2, NZ, CP), jnp.float32),
    mesh=_mesh,
    compiler_params=pltpu.CompilerParams(use_tc_tiling_on_sc=False),
    scratch_types=[
        pltpu.VMEM((NB, BB), jnp.int32),
        pltpu.VMEM((NB, BB), jnp.int32),
        pltpu.VMEM((BB, CP), jnp.float32),
        pltpu.VMEM((BB, CP), jnp.float32),
        pltpu.VMEM((BB, CP), jnp.float32),
        pltpu.VMEM((BB, CP), jnp.float32),
        pltpu.VMEM_SHARED((NZ, CP), jnp.float32),
        pltpu.VMEM_SHARED((NZ, CP), jnp.float32),
        pltpu.SemaphoreType.DMA,
        pltpu.SemaphoreType.DMA,
        pltpu.SemaphoreType.DMA,
        pltpu.SemaphoreType.DMA,
        pltpu.SemaphoreType.DMA,
        pltpu.SemaphoreType.DMA,
        pltpu.SemaphoreType.DMA,
        pltpu.SemaphoreType.DMA,
    ],
)
def _hop_kernel(y_hbm, rs_hbm, cs_hbm, zeros_hbm, out_hbm,
                ridx, cidx, gb0, gb1, gb2, gb3, z_sp, y_sp,
                gs0, gs1, gs2, gs3, ss0, ss1, ss2, ss3):
    c = lax.axis_index("c")
    s = lax.axis_index("s")
    wid = s * 2 + c
    gbufs = (gb0, gb1, gb2, gb3)
    gsems = (gs0, gs1, gs2, gs3)
    ssems = (ss0, ss1, ss2, ss3)
    pltpu.sync_copy(zeros_hbm.at[pl.ds(s * RPT, RPT)], z_sp.at[pl.ds(s * RPT, RPT)])
    pltpu.sync_copy(y_hbm.at[pl.ds(s * RPT, RPT)], y_sp.at[pl.ds(s * RPT, RPT)])
    pltpu.sync_copy(rs_hbm.at[wid], ridx)
    pltpu.sync_copy(cs_hbm.at[wid], cidx)
    plsc.subcore_barrier()

    pltpu.async_copy(y_sp.at[ridx.at[0]], gb0, gs0)
    pltpu.async_copy(y_sp.at[ridx.at[1]], gb1, gs1)

    def group(i, carry):
        j0 = 4 * i
        for b in range(4):
            j = j0 + b

            @pl.when(j >= 2)
            def _():
                pltpu.make_async_copy(gbufs[(b + 2) % 4],
                                      z_sp.at[cidx.at[j - 2]],
                                      ssems[(b + 2) % 4]).wait()

            pltpu.make_async_copy(y_sp.at[ridx.at[j]], gbufs[b], gsems[b]).wait()
            pltpu.async_copy(gbufs[b], z_sp.at[cidx.at[j]], ssems[b], add=True)

            @pl.when(j + 2 < NB)
            def _():
                pltpu.async_copy(y_sp.at[ridx.at[j + 2]],
                                 gbufs[(b + 2) % 4], gsems[(b + 2) % 4])
        return carry

    lax.fori_loop(0, NB // 4, group, 0)
    pltpu.make_async_copy(gb2, z_sp.at[cidx.at[NB - 2]], ss2).wait()
    pltpu.make_async_copy(gb3, z_sp.at[cidx.at[NB - 1]], ss3).wait()
    plsc.subcore_barrier()
    pltpu.sync_copy(z_sp.at[pl.ds(s * RPT, RPT)], out_hbm.at[c, pl.ds(s * RPT, RPT)])


def _dense_body(x_ref, w1_ref, b1_ref, w2_ref, b2_ref, o_ref):
    h = jnp.dot(x_ref[...], w1_ref[...], preferred_element_type=jnp.float32)
    h = jnp.maximum(h + b1_ref[...], 0.0)
    o_ref[...] = jnp.dot(h, w2_ref[...], preferred_element_type=jnp.float32) + b2_ref[...]


def _final_body(pps_ref, wp_ref, bp_ref, o_ref):
    pps = pps_ref[...]
    wp = wp_ref[...]
    bp = bp_ref[0, 0]
    score = jnp.sum(pps * wp[0][None, None, :], axis=2) + bp
    retain = jax.nn.sigmoid(score)
    out = jnp.sum(retain[:, :, None] * pps, axis=0)
    colp = lax.broadcasted_iota(jnp.int32, out.shape, 1)
    logits = jnp.where(colp < C, out, -jnp.inf)
    m = jnp.max(logits, axis=1, keepdims=True)
    lse = jnp.log(jnp.sum(jnp.exp(logits - m), axis=1, keepdims=True))
    o_ref[...] = logits - m - lse


def kernel(x, edge_index, W1, b1, W2, b2, Wp, bp):
    f32 = jnp.float32
    x_pad = jnp.zeros((NZ, F_IN), f32).at[:N].set(x)
    W1t = W1.T
    W2t = jnp.zeros((HID, CP), f32).at[:, :C].set(W2.T)
    b1r = b1.reshape(1, HID)
    b2r = jnp.zeros((1, CP), f32).at[0, :C].set(b2)
    Wpr = jnp.zeros((1, CP), f32).at[0, :C].set(Wp[0])
    bpr = bp.reshape(1, 1)

    rs0 = edge_index[0]
    cs0 = edge_index[1]
    ar = jnp.arange(E, dtype=jnp.int32)
    trash = N + (ar % (NZ - N))
    cs1 = jnp.where(rs0 == cs0, trash, cs0)
    npad = E_PAD - E
    arp = jnp.arange(npad, dtype=jnp.int32)
    rs3 = jnp.concatenate([rs0, arp % N]).reshape(NW, NB, BB)
    cs3 = jnp.concatenate([cs1, N + (arp % (NZ - N))]).reshape(NW, NB, BB)

    zeros2 = jnp.zeros((NZ, CP), f32)
    zeros1 = jnp.zeros((NZ,), f32)
    ones1 = jnp.ones((BB,), f32)

    BD = 512
    h = pl.pallas_call(
        _dense_body,
        grid=(NZ // BD,),
        in_specs=[
            pl.BlockSpec((BD, F_IN), lambda i: (i, 0)),
            pl.BlockSpec((F_IN, HID), lambda i: (0, 0)),
            pl.BlockSpec((1, HID), lambda i: (0, 0)),
            pl.BlockSpec((HID, CP), lambda i: (0, 0)),
            pl.BlockSpec((1, CP), lambda i: (0, 0)),
        ],
        out_specs=pl.BlockSpec((BD, CP), lambda i: (i, 0)),
        out_shape=jax.ShapeDtypeStruct((NZ, CP), f32),
    )(x_pad, W1t, b1r, W2t, b2r)

    degp = _deg_kernel(cs3, zeros1, ones1)
    deg = degp[0] + degp[1] + 1.0
    disc = lax.rsqrt(deg)[:, None]

    preds = [h]
    y = h * disc
    for k in range(K_HOPS):
        zp = _hop_kernel(y, rs3, cs3, zeros2)
        cur = disc * (zp[0] + zp[1] + y)
        preds.append(cur)
        if k + 1 < K_HOPS:
            y = disc * cur

    pps = jnp.stack(preds)
    out = pl.pallas_call(
        _final_body,
        grid=(NZ // BD,),
        in_specs=[
            pl.BlockSpec((K_HOPS + 1, BD, CP), lambda i: (0, i, 0)),
            pl.BlockSpec((1, CP), lambda i: (0, 0)),
            pl.BlockSpec((1, 1), lambda i: (0, 0)),
        ],
        out_specs=pl.BlockSpec((BD, CP), lambda i: (i, 0)),
        out_shape=jax.ShapeDtypeStruct((NZ, CP), f32),
    )(pps, Wpr, bpr)
    return out[:N, :C]
